# baseline (speedup 1.0000x reference)
import functools

import jax
import jax.numpy as jnp
from jax import lax
from jax.experimental import pallas as pl
from jax.experimental.pallas import tpu as pltpu
from jax.experimental.pallas import tpu_sc as plsc

N_NODES = 100000
N_EDGES = 6400000
NC = 2
NS = 16
NW = NC * NS

N_PAD = 100352
ZSLICE = N_PAD // NS
CSLICE = N_PAD // NW
EPW = N_EDGES // NW
CHUNK = 4000
UNROLL = 5

_mesh = plsc.VectorSubcoreMesh(core_axis_name="c", subcore_axis_name="s")
_params = pltpu.CompilerParams(needs_layout_passes=False)


@functools.partial(
    pl.kernel,
    out_type=jax.ShapeDtypeStruct((NC * N_PAD,), jnp.float32),
    mesh=_mesh,
    scratch_types=[
        pltpu.VMEM((CHUNK,), jnp.int32),
        pltpu.VMEM((CHUNK,), jnp.int32),
        pltpu.VMEM((CHUNK,), jnp.int32),
        pltpu.VMEM((CHUNK,), jnp.int32),
        pltpu.VMEM((CHUNK,), jnp.float32),
        pltpu.VMEM((CHUNK,), jnp.float32),
        pltpu.VMEM((N_NODES,), jnp.float32),
        pltpu.VMEM_SHARED((N_PAD,), jnp.float32),
        pltpu.SemaphoreType.DMA,
        pltpu.SemaphoreType.DMA,
        pltpu.SemaphoreType.DMA,
    ],
    compiler_params=_params,
)
def _scatter_accumulate(src_hbm, dst_hbm, power_hbm, out_hbm,
                        idx_s0, idx_s1, idx_d0, idx_d1, vals0, vals1,
                        power_v, acc_sh, lsem0, lsem1, ssem):
    idx_s = (idx_s0, idx_s1)
    idx_d = (idx_d0, idx_d1)
    vals = (vals0, vals1)
    lsem = (lsem0, lsem1)
    c = lax.axis_index("c")
    s = lax.axis_index("s")

    def zero_body(j, carry):
        power_v[pl.ds(j * 16, 16)] = jnp.zeros((16,), jnp.float32)
        return carry
    lax.fori_loop(0, ZSLICE // 16, zero_body, 0)
    pltpu.sync_copy(power_v.at[pl.ds(0, ZSLICE)],
                    acc_sh.at[pl.ds(s * ZSLICE, ZSLICE)])

    pltpu.sync_copy(power_hbm, power_v)

    plsc.subcore_barrier()

    base = c * (N_EDGES // NC) + s * EPW
    n_chunks = EPW // CHUNK
    n_vec = CHUNK // 16

    def edge_ds(i):
        return pl.ds(base + i * CHUNK, CHUNK)

    pend_l = {}
    pend_s = {}

    def start_load(i):
        b = i % 2
        pend_l[i] = (
            pltpu.async_copy(src_hbm.at[edge_ds(i)], idx_s[b], lsem[b]),
            pltpu.async_copy(dst_hbm.at[edge_ds(i)], idx_d[b], lsem[b]),
        )

    start_load(0)
    for i in range(n_chunks):
        b = i % 2
        d1, d2 = pend_l.pop(i)
        d1.wait()
        d2.wait()

        @plsc.parallel_loop(0, n_vec, step=1, unroll=UNROLL)
        def gather_body(j, b=b):
            sl = pl.ds(j * 16, 16)
            vals[b][sl] = plsc.load_gather(power_v, [idx_s[b][sl]])

        if i > 0:
            pend_s.pop(i - 1).wait()
        if i + 1 < n_chunks:
            start_load(i + 1)
        pend_s[i] = pltpu.async_copy(vals[b], acc_sh.at[idx_d[b]], ssem,
                                     add=True)

    pend_s.pop(n_chunks - 1).wait()
    plsc.subcore_barrier()

    pltpu.sync_copy(acc_sh.at[pl.ds(s * ZSLICE, ZSLICE)],
                    power_v.at[pl.ds(0, ZSLICE)])
    pltpu.sync_copy(power_v.at[pl.ds(0, ZSLICE)],
                    out_hbm.at[pl.ds(c * N_PAD + s * ZSLICE, ZSLICE)])


@functools.partial(
    pl.kernel,
    out_type=jax.ShapeDtypeStruct((N_PAD,), jnp.float32),
    mesh=_mesh,
    scratch_types=[
        pltpu.VMEM((CSLICE,), jnp.float32),
        pltpu.VMEM((CSLICE,), jnp.float32),
        pltpu.VMEM((16,), jnp.float32),
    ],
)
def _combine(parts_hbm, ts_hbm, out_hbm, a_v, b_v, ts_v):
    c = lax.axis_index("c")
    s = lax.axis_index("s")
    w = s * NC + c
    off = w * CSLICE
    pltpu.sync_copy(parts_hbm.at[pl.ds(off, CSLICE)], a_v)
    pltpu.sync_copy(parts_hbm.at[pl.ds(N_PAD + off, CSLICE)], b_v)
    pltpu.sync_copy(ts_hbm, ts_v)
    ts = ts_v[...]

    def body(j, carry):
        sl = pl.ds(j * 16, 16)
        a_v[sl] = (a_v[sl] + b_v[sl]) * ts
        return carry
    lax.fori_loop(0, CSLICE // 16, body, 0)
    pltpu.sync_copy(a_v, out_hbm.at[pl.ds(off, CSLICE)])


def kernel(power, time_step, edge_index):
    src = edge_index[0].astype(jnp.int32)
    dst = edge_index[1].astype(jnp.int32)
    ts16 = jnp.broadcast_to(time_step.astype(jnp.float32), (16,))
    parts = _scatter_accumulate(src, dst, power)
    out = _combine(parts, ts16)
    return out[:N_NODES]


# speedup vs baseline: 1.0859x; 1.0859x over previous
import functools

import jax
import jax.numpy as jnp
from jax import lax
from jax.experimental import pallas as pl
from jax.experimental.pallas import tpu as pltpu
from jax.experimental.pallas import tpu_sc as plsc

N_NODES = 100000
N_EDGES = 6400000
NC = 2
NS = 16
NW = NC * NS

N_PAD = 100352
ZSLICE = N_PAD // NS
CSLICE = N_PAD // NW
EPW = N_EDGES // NW
CHUNK = 4000
UNROLL = 5

_mesh = plsc.VectorSubcoreMesh(core_axis_name="c", subcore_axis_name="s")
_params = pltpu.CompilerParams(needs_layout_passes=False)


@functools.partial(
    pl.kernel,
    out_type=jax.ShapeDtypeStruct((NC * N_PAD,), jnp.float32),
    mesh=_mesh,
    scratch_types=[
        pltpu.VMEM((CHUNK,), jnp.int32),
        pltpu.VMEM((CHUNK,), jnp.int32),
        pltpu.VMEM((CHUNK,), jnp.int32),
        pltpu.VMEM((CHUNK,), jnp.int32),
        pltpu.VMEM((CHUNK,), jnp.float32),
        pltpu.VMEM((CHUNK,), jnp.float32),
        pltpu.VMEM((N_NODES,), jnp.float32),
        pltpu.VMEM_SHARED((N_PAD,), jnp.float32),
        pltpu.SemaphoreType.DMA,
        pltpu.SemaphoreType.DMA,
        pltpu.SemaphoreType.DMA,
    ],
    compiler_params=_params,
)
def _scatter_accumulate(edges_hbm, power_hbm, out_hbm,
                        idx_s0, idx_s1, idx_d0, idx_d1, vals0, vals1,
                        power_v, acc_sh, lsem0, lsem1, ssem):
    idx_s = (idx_s0, idx_s1)
    idx_d = (idx_d0, idx_d1)
    vals = (vals0, vals1)
    lsem = (lsem0, lsem1)
    c = lax.axis_index("c")
    s = lax.axis_index("s")

    def zero_body(j, carry):
        power_v[pl.ds(j * 16, 16)] = jnp.zeros((16,), jnp.float32)
        return carry
    lax.fori_loop(0, ZSLICE // 16, zero_body, 0)
    pltpu.sync_copy(power_v.at[pl.ds(0, ZSLICE)],
                    acc_sh.at[pl.ds(s * ZSLICE, ZSLICE)])

    pltpu.sync_copy(power_hbm, power_v)

    plsc.subcore_barrier()

    base = c * (N_EDGES // NC) + s * EPW
    n_chunks = EPW // CHUNK
    n_vec = CHUNK // 16

    pend_l = {}
    pend_s = {}

    def start_load(i):
        b = i % 2
        off = base + i * CHUNK
        pend_l[i] = (
            pltpu.async_copy(edges_hbm.at[pl.ds(off, CHUNK)],
                             idx_s[b], lsem[b]),
            pltpu.async_copy(edges_hbm.at[pl.ds(N_EDGES + off, CHUNK)],
                             idx_d[b], lsem[b]),
        )

    start_load(0)
    for i in range(n_chunks):
        b = i % 2
        d1, d2 = pend_l.pop(i)
        d1.wait()
        d2.wait()

        @plsc.parallel_loop(0, n_vec, step=1, unroll=UNROLL)
        def gather_body(j, b=b):
            sl = pl.ds(j * 16, 16)
            vals[b][sl] = plsc.load_gather(power_v, [idx_s[b][sl]])

        if i > 0:
            pend_s.pop(i - 1).wait()
        if i + 1 < n_chunks:
            start_load(i + 1)
        pend_s[i] = pltpu.async_copy(vals[b], acc_sh.at[idx_d[b]], ssem,
                                     add=True)

    pend_s.pop(n_chunks - 1).wait()
    plsc.subcore_barrier()

    pltpu.sync_copy(acc_sh.at[pl.ds(s * ZSLICE, ZSLICE)],
                    power_v.at[pl.ds(0, ZSLICE)])
    pltpu.sync_copy(power_v.at[pl.ds(0, ZSLICE)],
                    out_hbm.at[pl.ds(c * N_PAD + s * ZSLICE, ZSLICE)])


_CTAIL = N_NODES - (NW - 1) * CSLICE


@functools.partial(
    pl.kernel,
    out_type=jax.ShapeDtypeStruct((N_NODES,), jnp.float32),
    mesh=_mesh,
    scratch_types=[
        pltpu.VMEM((CSLICE,), jnp.float32),
        pltpu.VMEM((CSLICE,), jnp.float32),
        pltpu.VMEM((16,), jnp.float32),
    ],
)
def _combine(parts_hbm, ts_hbm, out_hbm, a_v, b_v, ts_v):
    c = lax.axis_index("c")
    s = lax.axis_index("s")
    w = s * NC + c
    off = w * CSLICE
    pltpu.sync_copy(ts_hbm, ts_v)

    def do(n):
        pltpu.sync_copy(parts_hbm.at[pl.ds(off, n)], a_v.at[pl.ds(0, n)])
        pltpu.sync_copy(parts_hbm.at[pl.ds(N_PAD + off, n)],
                        b_v.at[pl.ds(0, n)])
        ts = ts_v[...]

        def body(j, carry):
            sl = pl.ds(j * 16, 16)
            a_v[sl] = (a_v[sl] + b_v[sl]) * ts
            return carry
        lax.fori_loop(0, n // 16, body, 0)
        pltpu.sync_copy(a_v.at[pl.ds(0, n)], out_hbm.at[pl.ds(off, n)])

    @pl.when(w < NW - 1)
    def _():
        do(CSLICE)

    @pl.when(w == NW - 1)
    def _():
        do(_CTAIL)


def kernel(power, time_step, edge_index):
    edges = edge_index.astype(jnp.int32).reshape(2 * N_EDGES)
    ts16 = jnp.broadcast_to(time_step.astype(jnp.float32), (16,))
    parts = _scatter_accumulate(edges, power)
    return _combine(parts, ts16)


# speedup vs baseline: 1.1578x; 1.0662x over previous
import functools

import jax
import jax.numpy as jnp
from jax import lax
from jax.experimental import pallas as pl
from jax.experimental.pallas import tpu as pltpu
from jax.experimental.pallas import tpu_sc as plsc

N_NODES = 100000
N_EDGES = 6400000
NC = 2
NS = 16
NW = NC * NS

N_PAD = 100352
ZSLICE = N_PAD // NS
CSLICE = N_PAD // NW
CHUNK = 2560
N_CHUNKS = N_EDGES // CHUNK
CPT = N_CHUNKS // NW
N_EXTRA = N_CHUNKS - CPT * NW
UNROLL = 5

_mesh = plsc.VectorSubcoreMesh(core_axis_name="c", subcore_axis_name="s")
_params = pltpu.CompilerParams(needs_layout_passes=False)


@functools.partial(
    pl.kernel,
    out_type=jax.ShapeDtypeStruct((NC * N_PAD,), jnp.float32),
    mesh=_mesh,
    scratch_types=[
        pltpu.VMEM((2, CHUNK), jnp.int32),
        pltpu.VMEM((2, CHUNK), jnp.int32),
        pltpu.VMEM((CHUNK,), jnp.int32),
        pltpu.VMEM((CHUNK,), jnp.int32),
        pltpu.VMEM((CHUNK,), jnp.float32),
        pltpu.VMEM((CHUNK,), jnp.float32),
        pltpu.VMEM((N_NODES,), jnp.float32),
        pltpu.VMEM_SHARED((N_PAD,), jnp.float32),
        pltpu.SemaphoreType.DMA,
        pltpu.SemaphoreType.DMA,
        pltpu.SemaphoreType.DMA,
    ],
    compiler_params=_params,
)
def _scatter_accumulate(edges_hbm, power_hbm, out_hbm,
                        idx0, idx1, idxd0, idxd1, vals0, vals1,
                        power_v, acc_sh, lsem0, lsem1, ssem):
    idx = (idx0, idx1)
    idxd = (idxd0, idxd1)
    vals = (vals0, vals1)
    lsem = (lsem0, lsem1)
    c = lax.axis_index("c")
    s = lax.axis_index("s")
    w = s * NC + c

    def zero_body(j, carry):
        power_v[pl.ds(j * 16, 16)] = jnp.zeros((16,), jnp.float32)
        return carry
    lax.fori_loop(0, ZSLICE // 16, zero_body, 0)
    pltpu.sync_copy(power_v.at[pl.ds(0, ZSLICE)],
                    acc_sh.at[pl.ds(s * ZSLICE, ZSLICE)])

    pltpu.sync_copy(power_hbm, power_v)

    plsc.subcore_barrier()

    n_vec = CHUNK // 16

    pend_l = {}
    pend_s = {}

    def start_load(i):
        b = i % 2
        pend_l[i] = pltpu.async_copy(
            edges_hbm.at[:, pl.ds((w + i * NW) * CHUNK, CHUNK)],
            idx[b], lsem[b])

    def gather_chunk(b):
        @plsc.parallel_loop(0, n_vec, step=1, unroll=UNROLL)
        def gather_body(j):
            sl = pl.ds(j * 16, 16)
            vals[b][sl] = plsc.load_gather(power_v, [idx[b][0, sl]])
            idxd[b][sl] = idx[b][1, sl]

    start_load(0)
    for i in range(CPT):
        b = i % 2
        pend_l.pop(i).wait()
        gather_chunk(b)
        if i > 0:
            pend_s.pop(i - 1).wait()
        if i + 1 < CPT:
            start_load(i + 1)
        pend_s[i] = pltpu.async_copy(vals[b], acc_sh.at[idxd[b]], ssem,
                                     add=True)

    pend_s.pop(CPT - 1).wait()

    @pl.when(w < N_EXTRA)
    def _():
        pltpu.sync_copy(
            edges_hbm.at[:, pl.ds((w + CPT * NW) * CHUNK, CHUNK)], idx0)
        gather_chunk(0)
        pltpu.sync_copy(vals0, acc_sh.at[idxd0], add=True)

    plsc.subcore_barrier()

    pltpu.sync_copy(acc_sh.at[pl.ds(s * ZSLICE, ZSLICE)],
                    power_v.at[pl.ds(0, ZSLICE)])
    pltpu.sync_copy(power_v.at[pl.ds(0, ZSLICE)],
                    out_hbm.at[pl.ds(c * N_PAD + s * ZSLICE, ZSLICE)])


_CTAIL = N_NODES - (NW - 1) * CSLICE


@functools.partial(
    pl.kernel,
    out_type=jax.ShapeDtypeStruct((N_NODES,), jnp.float32),
    mesh=_mesh,
    scratch_types=[
        pltpu.VMEM((CSLICE,), jnp.float32),
        pltpu.VMEM((CSLICE,), jnp.float32),
        pltpu.VMEM((16,), jnp.float32),
    ],
)
def _combine(parts_hbm, ts_hbm, out_hbm, a_v, b_v, ts_v):
    c = lax.axis_index("c")
    s = lax.axis_index("s")
    w = s * NC + c
    off = w * CSLICE
    pltpu.sync_copy(ts_hbm, ts_v)

    def do(n):
        pltpu.sync_copy(parts_hbm.at[pl.ds(off, n)], a_v.at[pl.ds(0, n)])
        pltpu.sync_copy(parts_hbm.at[pl.ds(N_PAD + off, n)],
                        b_v.at[pl.ds(0, n)])
        ts = ts_v[...]

        def body(j, carry):
            sl = pl.ds(j * 16, 16)
            a_v[sl] = (a_v[sl] + b_v[sl]) * ts
            return carry
        lax.fori_loop(0, n // 16, body, 0)
        pltpu.sync_copy(a_v.at[pl.ds(0, n)], out_hbm.at[pl.ds(off, n)])

    @pl.when(w < NW - 1)
    def _():
        do(CSLICE)

    @pl.when(w == NW - 1)
    def _():
        do(_CTAIL)


def kernel(power, time_step, edge_index):
    edges = edge_index.astype(jnp.int32)
    ts16 = jnp.broadcast_to(time_step.astype(jnp.float32), (16,))
    parts = _scatter_accumulate(edges, power)
    return _combine(parts, ts16)


# speedup vs baseline: 1.2530x; 1.0822x over previous
import functools

import jax
import jax.numpy as jnp
from jax import lax
from jax.experimental import pallas as pl
from jax.experimental.pallas import tpu as pltpu
from jax.experimental.pallas import tpu_sc as plsc

N_NODES = 100000
N_EDGES = 6400000
NC = 2
NS = 16
NW = NC * NS

N_PAD = 100352
ZSLICE = N_PAD // NS
CSLICE = N_PAD // NW
CHUNK = 3200
N_CHUNKS = N_EDGES // CHUNK
CPT = N_CHUNKS // NW
N_EXTRA = N_CHUNKS - CPT * NW
UNROLL = 5

_mesh = plsc.VectorSubcoreMesh(core_axis_name="c", subcore_axis_name="s")
_params = pltpu.CompilerParams(needs_layout_passes=False)


@functools.partial(
    pl.kernel,
    out_type=jax.ShapeDtypeStruct((NC * N_PAD,), jnp.float32),
    mesh=_mesh,
    scratch_types=[
        pltpu.VMEM((2, CHUNK), jnp.int32),
        pltpu.VMEM((CHUNK,), jnp.int32),
        pltpu.VMEM((CHUNK,), jnp.int32),
        pltpu.VMEM((CHUNK,), jnp.float32),
        pltpu.VMEM((CHUNK,), jnp.float32),
        pltpu.VMEM((N_NODES,), jnp.float32),
        pltpu.VMEM_SHARED((N_PAD,), jnp.float32),
        pltpu.SemaphoreType.DMA,
        pltpu.SemaphoreType.DMA,
    ],
    compiler_params=_params,
)
def _scatter_accumulate(edges_hbm, power_hbm, out_hbm,
                        idx2, idxd0, idxd1, vals0, vals1,
                        power_v, acc_sh, lsem, ssem):
    idxd = (idxd0, idxd1)
    vals = (vals0, vals1)
    c = lax.axis_index("c")
    s = lax.axis_index("s")
    w = s * NC + c

    def zero_body(j, carry):
        power_v[pl.ds(j * 16, 16)] = jnp.zeros((16,), jnp.float32)
        return carry
    lax.fori_loop(0, ZSLICE // 16, zero_body, 0)
    pltpu.sync_copy(power_v.at[pl.ds(0, ZSLICE)],
                    acc_sh.at[pl.ds(s * ZSLICE, ZSLICE)])

    pltpu.sync_copy(power_hbm, power_v)

    plsc.subcore_barrier()

    n_vec = CHUNK // 16

    pend_l = {}
    pend_s = {}

    def start_load(i):
        pend_l[i] = pltpu.async_copy(
            edges_hbm.at[:, pl.ds((w + i * NW) * CHUNK, CHUNK)],
            idx2, lsem)

    def gather_chunk(b):
        @plsc.parallel_loop(0, n_vec, step=1, unroll=UNROLL)
        def gather_body(j):
            sl = pl.ds(j * 16, 16)
            vals[b][sl] = plsc.load_gather(power_v, [idx2[0, sl]])
            idxd[b][sl] = idx2[1, sl]

    start_load(0)
    for i in range(CPT):
        b = i % 2
        pend_l.pop(i).wait()
        gather_chunk(b)
        if i + 1 < CPT:
            start_load(i + 1)
        if i > 0:
            pend_s.pop(i - 1).wait()
        pend_s[i] = pltpu.async_copy(vals[b], acc_sh.at[idxd[b]], ssem,
                                     add=True)

    pend_s.pop(CPT - 1).wait()

    @pl.when(w < N_EXTRA)
    def _():
        pltpu.sync_copy(
            edges_hbm.at[:, pl.ds((w + CPT * NW) * CHUNK, CHUNK)], idx2)
        gather_chunk(0)
        pltpu.sync_copy(vals0, acc_sh.at[idxd0], add=True)

    plsc.subcore_barrier()

    pltpu.sync_copy(acc_sh.at[pl.ds(s * ZSLICE, ZSLICE)],
                    power_v.at[pl.ds(0, ZSLICE)])
    pltpu.sync_copy(power_v.at[pl.ds(0, ZSLICE)],
                    out_hbm.at[pl.ds(c * N_PAD + s * ZSLICE, ZSLICE)])


_CTAIL = N_NODES - (NW - 1) * CSLICE


@functools.partial(
    pl.kernel,
    out_type=jax.ShapeDtypeStruct((N_NODES,), jnp.float32),
    mesh=_mesh,
    scratch_types=[
        pltpu.VMEM((CSLICE,), jnp.float32),
        pltpu.VMEM((CSLICE,), jnp.float32),
        pltpu.VMEM((16,), jnp.float32),
    ],
)
def _combine(parts_hbm, ts_hbm, out_hbm, a_v, b_v, ts_v):
    c = lax.axis_index("c")
    s = lax.axis_index("s")
    w = s * NC + c
    off = w * CSLICE
    pltpu.sync_copy(ts_hbm, ts_v)

    def do(n):
        pltpu.sync_copy(parts_hbm.at[pl.ds(off, n)], a_v.at[pl.ds(0, n)])
        pltpu.sync_copy(parts_hbm.at[pl.ds(N_PAD + off, n)],
                        b_v.at[pl.ds(0, n)])
        ts = ts_v[...]

        def body(j, carry):
            sl = pl.ds(j * 16, 16)
            a_v[sl] = (a_v[sl] + b_v[sl]) * ts
            return carry
        lax.fori_loop(0, n // 16, body, 0)
        pltpu.sync_copy(a_v.at[pl.ds(0, n)], out_hbm.at[pl.ds(off, n)])

    @pl.when(w < NW - 1)
    def _():
        do(CSLICE)

    @pl.when(w == NW - 1)
    def _():
        do(_CTAIL)


def kernel(power, time_step, edge_index):
    edges = edge_index.astype(jnp.int32)
    ts16 = jnp.broadcast_to(time_step.astype(jnp.float32), (16,))
    parts = _scatter_accumulate(edges, power)
    return _combine(parts, ts16)
